# Initial kernel scaffold; baseline (speedup 1.0000x reference)
import jax
import jax.numpy as jnp
from jax.experimental import pallas as pl


def kernel(inputs, pos_table):
    raise NotImplementedError("write your pallas kernel here")


# speedup vs baseline: 1.6867x; 1.6867x over previous
import jax
import jax.numpy as jnp
from jax.experimental import pallas as pl

_BS = 512


def _body(x_ref, p_ref, o_ref):
    x = x_ref[0]
    p = p_ref[...]
    o_ref[0] = x + jnp.where(x != 0.0, p, 0.0)


def kernel(inputs, pos_table):
    B, L, D = inputs.shape
    return pl.pallas_call(
        _body,
        grid=(L // _BS, B),
        in_specs=[
            pl.BlockSpec((1, _BS, D), lambda s, b: (b, s, 0)),
            pl.BlockSpec((_BS, D), lambda s, b: (s, 0)),
        ],
        out_specs=pl.BlockSpec((1, _BS, D), lambda s, b: (b, s, 0)),
        out_shape=jax.ShapeDtypeStruct((B, L, D), inputs.dtype),
    )(inputs, pos_table)


# speedup vs baseline: 1.8747x; 1.1114x over previous
import jax
import jax.numpy as jnp
from jax.experimental import pallas as pl

_BS = 1024


def _body(x_ref, p_ref, o_ref):
    x = x_ref[0]
    p = p_ref[...]
    o_ref[0] = x + jnp.where(x != 0.0, p, 0.0)


def kernel(inputs, pos_table):
    B, L, D = inputs.shape
    return pl.pallas_call(
        _body,
        grid=(L // _BS, B),
        in_specs=[
            pl.BlockSpec((1, _BS, D), lambda s, b: (b, s, 0)),
            pl.BlockSpec((_BS, D), lambda s, b: (s, 0)),
        ],
        out_specs=pl.BlockSpec((1, _BS, D), lambda s, b: (b, s, 0)),
        out_shape=jax.ShapeDtypeStruct((B, L, D), inputs.dtype),
    )(inputs, pos_table)


# speedup vs baseline: 1.9861x; 1.0594x over previous
import jax
import jax.numpy as jnp
from jax.experimental import pallas as pl

_BS = 2048


def _body(x_ref, p_ref, o_ref):
    x = x_ref[0]
    p = p_ref[...]
    o_ref[0] = x + jnp.where(x != 0.0, p, 0.0)


def kernel(inputs, pos_table):
    B, L, D = inputs.shape
    return pl.pallas_call(
        _body,
        grid=(L // _BS, B),
        in_specs=[
            pl.BlockSpec((1, _BS, D), lambda s, b: (b, s, 0)),
            pl.BlockSpec((_BS, D), lambda s, b: (s, 0)),
        ],
        out_specs=pl.BlockSpec((1, _BS, D), lambda s, b: (b, s, 0)),
        out_shape=jax.ShapeDtypeStruct((B, L, D), inputs.dtype),
    )(inputs, pos_table)


# speedup vs baseline: 1.9866x; 1.0002x over previous
import jax
import jax.numpy as jnp
from jax.experimental import pallas as pl
from jax.experimental.pallas import tpu as pltpu

_BS = 2048


def _body(x_ref, p_ref, o_ref):
    x = x_ref[0]
    p = p_ref[...]
    o_ref[0] = x + jnp.where(x != 0.0, p, 0.0)


def kernel(inputs, pos_table):
    B, L, D = inputs.shape
    return pl.pallas_call(
        _body,
        grid=(L // _BS, B),
        in_specs=[
            pl.BlockSpec((1, _BS, D), lambda s, b: (b, s, 0)),
            pl.BlockSpec((_BS, D), lambda s, b: (s, 0)),
        ],
        out_specs=pl.BlockSpec((1, _BS, D), lambda s, b: (b, s, 0)),
        out_shape=jax.ShapeDtypeStruct((B, L, D), inputs.dtype),
        compiler_params=pltpu.CompilerParams(
            dimension_semantics=("parallel", "arbitrary"),
        ),
    )(inputs, pos_table)
